# Initial kernel scaffold; baseline (speedup 1.0000x reference)
import jax
import jax.numpy as jnp
from jax.experimental import pallas as pl


def kernel(d_mat_lon, d_mat_lat, table_lon, table_lat):
    raise NotImplementedError("write your pallas kernel here")


# speedup vs baseline: 2.7982x; 2.7982x over previous
import functools
import math

import jax
import jax.numpy as jnp
from jax import lax
from jax.experimental import pallas as pl
from jax.experimental.pallas import tpu as pltpu
from jax.experimental.pallas import tpu_sc as plsc

RADIUS_EARTH = 6371.0
MIN_DIST = 1.0
MAX_DIST = 1000.0
N_DIST = 1024
EMB_DIM = 16
LOG_MIN = math.log(MIN_DIST / RADIUS_EARTH)
LOG_MAX = math.log(MAX_DIST / RADIUS_EARTH)

_N = 2048
_B = _N * _N
_ROW_BLK = 128

_NC = 2
_NS = 16
_NW = _NC * _NS
_BPW = _B // _NW
_GW = 128
_CHUNK = 1024
_NGW = _CHUNK // _GW
_NCHUNK = _BPW // _CHUNK


def _bucket_idx(d):
    sgn = jnp.sign(d)
    t = jnp.log(jnp.abs(d))
    t = jnp.clip(t, LOG_MIN, LOG_MAX)
    t = (t - LOG_MIN) / (LOG_MAX - LOG_MIN)
    t = t * (N_DIST / 2.0)
    half = N_DIST / 2.0
    t = jnp.where(sgn < 0, t + (half - 1.0), half - 1.0 - t)
    idx = t.astype(jnp.int32)
    idx = jnp.where(idx < 0, idx + (N_DIST + 1), idx)
    return idx


def _idx_body(dlon_ref, dlat_ref, ilon_ref, ilat_ref):
    ilon_ref[...] = _bucket_idx(dlon_ref[...])
    ilat_ref[...] = _bucket_idx(dlat_ref[...])


def _compute_indices(d_mat_lon, d_mat_lat):
    spec = pl.BlockSpec((_ROW_BLK, _N), lambda i: (i, 0))
    return pl.pallas_call(
        _idx_body,
        grid=(_N // _ROW_BLK,),
        in_specs=[spec, spec],
        out_specs=[spec, spec],
        out_shape=[
            jax.ShapeDtypeStruct((_N, _N), jnp.int32),
            jax.ShapeDtypeStruct((_N, _N), jnp.int32),
        ],
    )(d_mat_lon, d_mat_lat)


def _sc_gather_sum(table_lon, table_lat, idx_lon, idx_lat):
    mesh = plsc.VectorSubcoreMesh(core_axis_name="c", subcore_axis_name="s")

    @functools.partial(
        pl.kernel,
        out_type=jax.ShapeDtypeStruct((_B, EMB_DIM), jnp.float32),
        mesh=mesh,
        compiler_params=pltpu.CompilerParams(use_tc_tiling_on_sc=False),
        scratch_types=[
            pltpu.VMEM((_NGW, _GW), jnp.int32),
            pltpu.VMEM((_NGW, _GW), jnp.int32),
            pltpu.VMEM((_CHUNK, EMB_DIM), jnp.float32),
            pltpu.VMEM((_CHUNK, EMB_DIM), jnp.float32),
            pltpu.SemaphoreType.DMA,
            pltpu.SemaphoreType.DMA,
        ],
    )
    def k(tlon_hbm, tlat_hbm, ilon_hbm, ilat_hbm, out_hbm,
          ilon_v, ilat_v, rlon_v, rlat_v, sem_lon, sem_lat):
        wid = lax.axis_index("s") * _NC + lax.axis_index("c")
        base = wid * _BPW

        @pl.loop(0, _NCHUNK)
        def _chunk(c):
            off = pl.multiple_of(base + c * _CHUNK, _CHUNK)
            irow = pl.multiple_of(off // _GW, _NGW)
            pltpu.sync_copy(ilon_hbm.at[pl.ds(irow, _NGW)], ilon_v)
            pltpu.sync_copy(ilat_hbm.at[pl.ds(irow, _NGW)], ilat_v)
            copies = []
            for j in range(_NGW):
                dst = rlon_v.at[pl.ds(j * _GW, _GW)]
                copies.append(
                    pltpu.async_copy(tlon_hbm.at[ilon_v.at[j]], dst, sem_lon))
            for j in range(_NGW):
                dst = rlat_v.at[pl.ds(j * _GW, _GW)]
                copies.append(
                    pltpu.async_copy(tlat_hbm.at[ilat_v.at[j]], dst, sem_lat))
            for cp in copies:
                cp.wait()

            @pl.loop(0, _CHUNK, step=8)
            def _add(r):
                for u in range(8):
                    plsc.addupdate(rlon_v.at[r + u], rlat_v[r + u])

            pltpu.sync_copy(rlon_v, out_hbm.at[pl.ds(off, _CHUNK)])

    return k(table_lon, table_lat, idx_lon, idx_lat)


def kernel(d_mat_lon, d_mat_lat, table_lon, table_lat):
    idx_lon, idx_lat = _compute_indices(d_mat_lon, d_mat_lat)
    idx_lon = idx_lon.reshape(_B // _GW, _GW)
    idx_lat = idx_lat.reshape(_B // _GW, _GW)
    flat = _sc_gather_sum(table_lon, table_lat, idx_lon, idx_lat)
    return flat.reshape(_N, _N, EMB_DIM)


# speedup vs baseline: 12.8148x; 4.5797x over previous
import functools
import math

import jax
import jax.numpy as jnp
from jax import lax
from jax.experimental import pallas as pl
from jax.experimental.pallas import tpu as pltpu
from jax.experimental.pallas import tpu_sc as plsc

RADIUS_EARTH = 6371.0
MIN_DIST = 1.0
MAX_DIST = 1000.0
N_DIST = 1024
EMB_DIM = 16
LOG_MIN = math.log(MIN_DIST / RADIUS_EARTH)
LOG_MAX = math.log(MAX_DIST / RADIUS_EARTH)

_N = 2048
_B = _N * _N
_ROW_BLK = 128
_V = N_DIST + 1
_TFLAT = EMB_DIM * _V

_NC = 2
_NS = 16
_NW = _NC * _NS
_BPW = _B // _NW
_CHUNK = 1024
_GRP = _CHUNK // 16
_NCHUNK = _BPW // _CHUNK


def _bucket_idx(d):
    sgn = jnp.sign(d)
    t = jnp.log(jnp.abs(d))
    t = jnp.clip(t, LOG_MIN, LOG_MAX)
    t = (t - LOG_MIN) / (LOG_MAX - LOG_MIN)
    t = t * (N_DIST / 2.0)
    half = N_DIST / 2.0
    t = jnp.where(sgn < 0, t + (half - 1.0), half - 1.0 - t)
    idx = t.astype(jnp.int32)
    idx = jnp.where(idx < 0, idx + (N_DIST + 1), idx)
    return idx


def _idx_body(dlon_ref, dlat_ref, ilon_ref, ilat_ref):
    ilon_ref[...] = _bucket_idx(dlon_ref[...])
    ilat_ref[...] = _bucket_idx(dlat_ref[...])


def _compute_indices(d_mat_lon, d_mat_lat):
    spec = pl.BlockSpec((_ROW_BLK, _N), lambda i: (i, 0))
    return pl.pallas_call(
        _idx_body,
        grid=(_N // _ROW_BLK,),
        in_specs=[spec, spec],
        out_specs=[spec, spec],
        out_shape=[
            jax.ShapeDtypeStruct((_N, _N), jnp.int32),
            jax.ShapeDtypeStruct((_N, _N), jnp.int32),
        ],
    )(d_mat_lon, d_mat_lat)


def _sc_gather_sum(tlon_flat, tlat_flat, idx_lon, idx_lat):
    mesh = plsc.VectorSubcoreMesh(core_axis_name="c", subcore_axis_name="s")

    @functools.partial(
        pl.kernel,
        out_type=jax.ShapeDtypeStruct((_B * EMB_DIM,), jnp.float32),
        mesh=mesh,
        compiler_params=pltpu.CompilerParams(use_tc_tiling_on_sc=False,
                                             needs_layout_passes=False),
        scratch_types=[
            pltpu.VMEM((_TFLAT,), jnp.float32),
            pltpu.VMEM((_TFLAT,), jnp.float32),
            pltpu.VMEM((2, _CHUNK), jnp.int32),
            pltpu.VMEM((2, _CHUNK), jnp.int32),
            pltpu.VMEM((2, _CHUNK * EMB_DIM), jnp.float32),
            pltpu.SemaphoreType.DMA,
            pltpu.SemaphoreType.DMA,
            pltpu.SemaphoreType.DMA,
            pltpu.SemaphoreType.DMA,
        ],
    )
    def k(tlon_hbm, tlat_hbm, ilon_hbm, ilat_hbm, out_hbm,
          tlon_v, tlat_v, ilon_v, ilat_v, obuf, si0, si1, so0, so1):
        si = (si0, si1)
        so = (so0, so1)
        wid = lax.axis_index("s") * _NC + lax.axis_index("c")
        base = wid * _BPW

        pltpu.sync_copy(tlon_hbm, tlon_v)
        pltpu.sync_copy(tlat_hbm, tlat_v)

        viota16 = lax.broadcasted_iota(jnp.int32, (16,), 0) * EMB_DIM

        def idx_src(c):
            off = pl.multiple_of(base + c * _CHUNK, _CHUNK)
            return (ilon_hbm.at[pl.ds(off, _CHUNK)],
                    ilat_hbm.at[pl.ds(off, _CHUNK)])

        def out_dst(c):
            off = pl.multiple_of((base + c * _CHUNK) * EMB_DIM,
                                 _CHUNK * EMB_DIM)
            return out_hbm.at[pl.ds(off, _CHUNK * EMB_DIM)]

        for b in (0, 1):
            slon, slat = idx_src(b)
            pltpu.async_copy(slon, ilon_v.at[b], si[b])
            pltpu.async_copy(slat, ilat_v.at[b], si[b])

        @pl.loop(0, _NCHUNK, step=2)
        def _pair(c0):
            for b in (0, 1):
                c = c0 + b
                slon, slat = idx_src(c)
                pltpu.make_async_copy(slon, ilon_v.at[b], si[b]).wait()
                pltpu.make_async_copy(slat, ilat_v.at[b], si[b]).wait()

                @pl.when(c >= 2)
                def _drain():
                    pltpu.make_async_copy(obuf.at[b], out_dst(c - 2),
                                          so[b]).wait()

                @pl.loop(0, _GRP)
                def _grp(g):
                    ivl = ilon_v.at[b][pl.ds(g * 16, 16)]
                    ivt = ilat_v.at[b][pl.ds(g * 16, 16)]
                    gbase = g * (16 * EMB_DIM)
                    for d in range(EMB_DIM):
                        gl = plsc.load_gather(tlon_v, [ivl + d * _V])
                        gt = plsc.load_gather(tlat_v, [ivt + d * _V])
                        sa = viota16 + (gbase + d)
                        plsc.store_scatter(obuf.at[b], [sa], gl + gt)

                pltpu.async_copy(obuf.at[b], out_dst(c), so[b])

                @pl.when(c + 2 < _NCHUNK)
                def _prefetch():
                    nlon, nlat = idx_src(c + 2)
                    pltpu.async_copy(nlon, ilon_v.at[b], si[b])
                    pltpu.async_copy(nlat, ilat_v.at[b], si[b])

        for b in (0, 1):
            pltpu.make_async_copy(obuf.at[b], out_dst(_NCHUNK - 2 + b),
                                  so[b]).wait()

    return k(tlon_flat, tlat_flat, idx_lon, idx_lat)


def kernel(d_mat_lon, d_mat_lat, table_lon, table_lat):
    idx_lon, idx_lat = _compute_indices(d_mat_lon, d_mat_lat)
    tlon_flat = jnp.transpose(table_lon).reshape(-1)
    tlat_flat = jnp.transpose(table_lat).reshape(-1)
    flat = _sc_gather_sum(tlon_flat, tlat_flat,
                          idx_lon.reshape(-1), idx_lat.reshape(-1))
    return flat.reshape(_N, _N, EMB_DIM)


# speedup vs baseline: 14.8956x; 1.1624x over previous
import functools
import math

import jax
import jax.numpy as jnp
from jax import lax
from jax.experimental import pallas as pl
from jax.experimental.pallas import tpu as pltpu
from jax.experimental.pallas import tpu_sc as plsc

RADIUS_EARTH = 6371.0
MIN_DIST = 1.0
MAX_DIST = 1000.0
N_DIST = 1024
EMB_DIM = 16
LOG_MIN = math.log(MIN_DIST / RADIUS_EARTH)
LOG_MAX = math.log(MAX_DIST / RADIUS_EARTH)

_N = 2048
_B = _N * _N
_ROW_BLK = 128
_V = N_DIST + 1
_TFLAT = EMB_DIM * _V

_NC = 2
_NS = 16
_NW = _NC * _NS
_BPW = _B // _NW
_CHUNK = 2048
_GRP = _CHUNK // 16
_NCHUNK = _BPW // _CHUNK
_PAD = EMB_DIM + 1


def _bucket_idx(d):
    sgn = jnp.sign(d)
    t = jnp.log(jnp.abs(d))
    t = jnp.clip(t, LOG_MIN, LOG_MAX)
    t = (t - LOG_MIN) / (LOG_MAX - LOG_MIN)
    t = t * (N_DIST / 2.0)
    half = N_DIST / 2.0
    t = jnp.where(sgn < 0, t + (half - 1.0), half - 1.0 - t)
    idx = t.astype(jnp.int32)
    idx = jnp.where(idx < 0, idx + (N_DIST + 1), idx)
    return idx


def _idx_body(dlon_ref, dlat_ref, ilon_ref, ilat_ref):
    ilon_ref[...] = _bucket_idx(dlon_ref[...])
    ilat_ref[...] = _bucket_idx(dlat_ref[...])


def _compute_indices(d_mat_lon, d_mat_lat):
    spec = pl.BlockSpec((_ROW_BLK, _N), lambda i: (i, 0))
    return pl.pallas_call(
        _idx_body,
        grid=(_N // _ROW_BLK,),
        in_specs=[spec, spec],
        out_specs=[spec, spec],
        out_shape=[
            jax.ShapeDtypeStruct((_N, _N), jnp.int32),
            jax.ShapeDtypeStruct((_N, _N), jnp.int32),
        ],
    )(d_mat_lon, d_mat_lat)


def _sc_gather_sum(tlon_flat, tlat_flat, idx_lon, idx_lat):
    mesh = plsc.VectorSubcoreMesh(core_axis_name="c", subcore_axis_name="s")

    @functools.partial(
        pl.kernel,
        out_type=jax.ShapeDtypeStruct((_B * EMB_DIM,), jnp.float32),
        mesh=mesh,
        compiler_params=pltpu.CompilerParams(use_tc_tiling_on_sc=False,
                                             needs_layout_passes=False),
        scratch_types=[
            pltpu.VMEM((_TFLAT,), jnp.float32),
            pltpu.VMEM((_TFLAT,), jnp.float32),
            pltpu.VMEM((2, _CHUNK), jnp.int32),
            pltpu.VMEM((2, _CHUNK), jnp.int32),
            pltpu.VMEM((2, _CHUNK * EMB_DIM), jnp.float32),
            pltpu.SemaphoreType.DMA,
            pltpu.SemaphoreType.DMA,
            pltpu.SemaphoreType.DMA,
            pltpu.SemaphoreType.DMA,
        ],
    )
    def k(tlon_hbm, tlat_hbm, ilon_hbm, ilat_hbm, out_hbm,
          tlon_v, tlat_v, ilon_v, ilat_v, obuf, si0, si1, so0, so1):
        si = (si0, si1)
        so = (so0, so1)
        wid = lax.axis_index("s") * _NC + lax.axis_index("c")
        base = wid * _BPW

        pltpu.sync_copy(tlon_hbm, tlon_v)
        pltpu.sync_copy(tlat_hbm, tlat_v)

        viota16 = lax.broadcasted_iota(jnp.int32, (16,), 0) * EMB_DIM

        def idx_src(c):
            off = pl.multiple_of(base + c * _CHUNK, _CHUNK)
            return (ilon_hbm.at[pl.ds(off, _CHUNK)],
                    ilat_hbm.at[pl.ds(off, _CHUNK)])

        def out_dst(c):
            off = pl.multiple_of((base + c * _CHUNK) * EMB_DIM,
                                 _CHUNK * EMB_DIM)
            return out_hbm.at[pl.ds(off, _CHUNK * EMB_DIM)]

        for b in (0, 1):
            slon, slat = idx_src(b)
            pltpu.async_copy(slon, ilon_v.at[b], si[b])
            pltpu.async_copy(slat, ilat_v.at[b], si[b])

        @pl.loop(0, _NCHUNK, step=2)
        def _pair(c0):
            for b in (0, 1):
                c = c0 + b
                slon, slat = idx_src(c)
                pltpu.make_async_copy(slon, ilon_v.at[b], si[b]).wait()
                pltpu.make_async_copy(slat, ilat_v.at[b], si[b]).wait()

                @pl.when(c >= 2)
                def _drain():
                    pltpu.make_async_copy(obuf.at[b], out_dst(c - 2),
                                          so[b]).wait()

                @plsc.parallel_loop(0, _GRP, 1, unroll=2)
                def _grp(g):
                    ivl = ilon_v.at[b][pl.ds(g * 16, 16)]
                    ivt = ilat_v.at[b][pl.ds(g * 16, 16)]
                    gbase = g * (16 * EMB_DIM)
                    for d in range(EMB_DIM):
                        gl = plsc.load_gather(tlon_v, [ivl + d * _V])
                        gt = plsc.load_gather(tlat_v, [ivt + d * _V])
                        sa = viota16 + (gbase + d)
                        plsc.store_scatter(obuf.at[b], [sa], gl + gt)

                pltpu.async_copy(obuf.at[b], out_dst(c), so[b])

                @pl.when(c + 2 < _NCHUNK)
                def _prefetch():
                    nlon, nlat = idx_src(c + 2)
                    pltpu.async_copy(nlon, ilon_v.at[b], si[b])
                    pltpu.async_copy(nlat, ilat_v.at[b], si[b])

        for b in (0, 1):
            pltpu.make_async_copy(obuf.at[b], out_dst(_NCHUNK - 2 + b),
                                  so[b]).wait()

    return k(tlon_flat, tlat_flat, idx_lon, idx_lat)


def kernel(d_mat_lon, d_mat_lat, table_lon, table_lat):
    idx_lon, idx_lat = _compute_indices(d_mat_lon, d_mat_lat)
    tlon_flat = jnp.transpose(table_lon).reshape(-1)
    tlat_flat = jnp.transpose(table_lat).reshape(-1)
    flat = _sc_gather_sum(tlon_flat, tlat_flat,
                          idx_lon.reshape(-1), idx_lat.reshape(-1))
    return flat.reshape(_N, _N, EMB_DIM)


# speedup vs baseline: 16.5807x; 1.1131x over previous
import functools
import math

import jax
import jax.numpy as jnp
from jax import lax
from jax.experimental import pallas as pl
from jax.experimental.pallas import tpu as pltpu
from jax.experimental.pallas import tpu_sc as plsc

RADIUS_EARTH = 6371.0
MIN_DIST = 1.0
MAX_DIST = 1000.0
N_DIST = 1024
EMB_DIM = 16
LOG_MIN = math.log(MIN_DIST / RADIUS_EARTH)
LOG_MAX = math.log(MAX_DIST / RADIUS_EARTH)

_N = 2048
_B = _N * _N
_ROW_BLK = 128
_V = N_DIST + 1
_TFLAT = EMB_DIM * _V

_NC = 2
_NS = 16
_NW = _NC * _NS
_BPW = _B // _NW
_CHUNK = 2048
_GRP = _CHUNK // 16
_NCHUNK = _BPW // _CHUNK
_PAD = EMB_DIM + 1


def _bucket_idx(d):
    sgn = jnp.sign(d)
    t = jnp.log(jnp.abs(d))
    t = jnp.clip(t, LOG_MIN, LOG_MAX)
    t = (t - LOG_MIN) / (LOG_MAX - LOG_MIN)
    t = t * (N_DIST / 2.0)
    half = N_DIST / 2.0
    t = jnp.where(sgn < 0, t + (half - 1.0), half - 1.0 - t)
    idx = t.astype(jnp.int32)
    idx = jnp.where(idx < 0, idx + (N_DIST + 1), idx)
    return idx


def _idx_body(dlon_ref, dlat_ref, ilon_ref, ilat_ref):
    ilon_ref[...] = _bucket_idx(dlon_ref[...])
    ilat_ref[...] = _bucket_idx(dlat_ref[...])


def _compute_indices(d_mat_lon, d_mat_lat):
    spec = pl.BlockSpec((_ROW_BLK, _N), lambda i: (i, 0))
    return pl.pallas_call(
        _idx_body,
        grid=(_N // _ROW_BLK,),
        in_specs=[spec, spec],
        out_specs=[spec, spec],
        out_shape=[
            jax.ShapeDtypeStruct((_N, _N), jnp.int32),
            jax.ShapeDtypeStruct((_N, _N), jnp.int32),
        ],
    )(d_mat_lon, d_mat_lat)


def _sc_gather_sum(tlon_flat, tlat_flat, idx_lon, idx_lat):
    mesh = plsc.VectorSubcoreMesh(core_axis_name="c", subcore_axis_name="s")

    @functools.partial(
        pl.kernel,
        out_type=jax.ShapeDtypeStruct((_B * EMB_DIM // 128, 128), jnp.float32),
        mesh=mesh,
        compiler_params=pltpu.CompilerParams(needs_layout_passes=False),
        scratch_types=[
            pltpu.VMEM((_TFLAT,), jnp.float32),
            pltpu.VMEM((_TFLAT,), jnp.float32),
            pltpu.VMEM((_CHUNK,), jnp.int32),
            pltpu.VMEM((_CHUNK,), jnp.int32),
            pltpu.VMEM((_CHUNK,), jnp.int32),
            pltpu.VMEM((_CHUNK,), jnp.int32),
            pltpu.VMEM((2, _CHUNK * EMB_DIM // 128, 128), jnp.float32),
            pltpu.SemaphoreType.DMA,
            pltpu.SemaphoreType.DMA,
            pltpu.SemaphoreType.DMA,
            pltpu.SemaphoreType.DMA,
        ],
    )
    def k(tlon_hbm, tlat_hbm, ilon_hbm, ilat_hbm, out_hbm,
          tlon_v, tlat_v, ilon0, ilon1, ilat0, ilat1, obuf,
          si0, si1, so0, so1):
        si = (si0, si1)
        so = (so0, so1)
        ilon = (ilon0, ilon1)
        ilat = (ilat0, ilat1)
        wid = lax.axis_index("s") * _NC + lax.axis_index("c")
        base = wid * _BPW

        pltpu.sync_copy(tlon_hbm, tlon_v)
        pltpu.sync_copy(tlat_hbm, tlat_v)

        iota = lax.broadcasted_iota(jnp.int32, (16,), 0)
        crow = iota >> 3
        ccol = (iota & 7) * EMB_DIM

        def idx_src(c):
            off = pl.multiple_of(base + c * _CHUNK, _CHUNK)
            return (ilon_hbm.at[pl.ds(off, _CHUNK)],
                    ilat_hbm.at[pl.ds(off, _CHUNK)])

        _OROW = _CHUNK * EMB_DIM // 128

        def out_dst(c):
            row = pl.multiple_of((base + c * _CHUNK) * EMB_DIM // 128, _OROW)
            return out_hbm.at[pl.ds(row, _OROW)]

        for b in (0, 1):
            slon, slat = idx_src(b)
            pltpu.async_copy(slon, ilon[b], si[b])
            pltpu.async_copy(slat, ilat[b], si[b])

        @pl.loop(0, _NCHUNK, step=2)
        def _pair(c0):
            for b in (0, 1):
                c = c0 + b
                slon, slat = idx_src(c)
                pltpu.make_async_copy(slon, ilon[b], si[b]).wait()
                pltpu.make_async_copy(slat, ilat[b], si[b]).wait()

                @pl.when(c >= 2)
                def _drain():
                    pltpu.make_async_copy(obuf.at[b], out_dst(c - 2),
                                          so[b]).wait()

                @plsc.parallel_loop(0, _GRP, 1, unroll=2)
                def _grp(g):
                    ivl = ilon[b][pl.ds(g * 16, 16)]
                    ivt = ilat[b][pl.ds(g * 16, 16)]
                    rowv = crow + g * 2
                    for d in range(EMB_DIM):
                        gl = plsc.load_gather(tlon_v, [ivl + d * _V])
                        gt = plsc.load_gather(tlat_v, [ivt + d * _V])
                        plsc.store_scatter(obuf.at[b], [rowv, ccol + d],
                                           gl + gt)

                pltpu.async_copy(obuf.at[b], out_dst(c), so[b])

                @pl.when(c + 2 < _NCHUNK)
                def _prefetch():
                    nlon, nlat = idx_src(c + 2)
                    pltpu.async_copy(nlon, ilon[b], si[b])
                    pltpu.async_copy(nlat, ilat[b], si[b])

        for b in (0, 1):
            pltpu.make_async_copy(obuf.at[b], out_dst(_NCHUNK - 2 + b),
                                  so[b]).wait()

    return k(tlon_flat, tlat_flat, idx_lon, idx_lat)


def kernel(d_mat_lon, d_mat_lat, table_lon, table_lat):
    idx_lon, idx_lat = _compute_indices(d_mat_lon, d_mat_lat)
    tlon_flat = jnp.transpose(table_lon).reshape(-1)
    tlat_flat = jnp.transpose(table_lat).reshape(-1)
    flat = _sc_gather_sum(tlon_flat, tlat_flat,
                          idx_lon.reshape(-1), idx_lat.reshape(-1))
    return flat.reshape(_N, _N, EMB_DIM)


# speedup vs baseline: 158.3515x; 9.5503x over previous
import functools
import math

import jax
import jax.numpy as jnp
from jax import lax
from jax.experimental import pallas as pl
from jax.experimental.pallas import tpu as pltpu
from jax.experimental.pallas import tpu_sc as plsc

RADIUS_EARTH = 6371.0
MIN_DIST = 1.0
MAX_DIST = 1000.0
N_DIST = 1024
EMB_DIM = 16
LOG_MIN = math.log(MIN_DIST / RADIUS_EARTH)
LOG_MAX = math.log(MAX_DIST / RADIUS_EARTH)

_N = 2048
_B = _N * _N
_ROW_BLK = 128
_V = N_DIST + 1
_TFLAT = EMB_DIM * _V

_NC = 2
_NS = 16
_NW = _NC * _NS
_BPW = _B // _NW
_CHUNK = 2048
_GRP = _CHUNK // 16
_NCHUNK = _BPW // _CHUNK
_PAD = EMB_DIM + 1


def _bucket_idx(d):
    sgn = jnp.sign(d)
    t = jnp.log(jnp.abs(d))
    t = jnp.clip(t, LOG_MIN, LOG_MAX)
    t = (t - LOG_MIN) / (LOG_MAX - LOG_MIN)
    t = t * (N_DIST / 2.0)
    half = N_DIST / 2.0
    t = jnp.where(sgn < 0, t + (half - 1.0), half - 1.0 - t)
    idx = t.astype(jnp.int32)
    idx = jnp.where(idx < 0, idx + (N_DIST + 1), idx)
    return idx


def _idx_body(dlon_ref, dlat_ref, ilon_ref, ilat_ref):
    ilon_ref[...] = _bucket_idx(dlon_ref[...])
    ilat_ref[...] = _bucket_idx(dlat_ref[...])


def _compute_indices(d_mat_lon, d_mat_lat):
    spec = pl.BlockSpec((_ROW_BLK, _N), lambda i: (i, 0))
    return pl.pallas_call(
        _idx_body,
        grid=(_N // _ROW_BLK,),
        in_specs=[spec, spec],
        out_specs=[spec, spec],
        out_shape=[
            jax.ShapeDtypeStruct((_N, _N), jnp.int32),
            jax.ShapeDtypeStruct((_N, _N), jnp.int32),
        ],
    )(d_mat_lon, d_mat_lat)


def _sc_gather_sum(tlon_flat, tlat_flat, idx_lon, idx_lat):
    mesh = plsc.VectorSubcoreMesh(core_axis_name="c", subcore_axis_name="s")

    @functools.partial(
        pl.kernel,
        out_type=jax.ShapeDtypeStruct((_N, EMB_DIM, _N), jnp.float32),
        mesh=mesh,
        compiler_params=pltpu.CompilerParams(needs_layout_passes=False),
        scratch_types=[
            pltpu.VMEM((_TFLAT,), jnp.float32),
            pltpu.VMEM((_TFLAT,), jnp.float32),
            pltpu.VMEM((_CHUNK,), jnp.int32),
            pltpu.VMEM((_CHUNK,), jnp.int32),
            pltpu.VMEM((_CHUNK,), jnp.int32),
            pltpu.VMEM((_CHUNK,), jnp.int32),
            pltpu.VMEM((2, EMB_DIM, _CHUNK), jnp.float32),
            pltpu.SemaphoreType.DMA,
            pltpu.SemaphoreType.DMA,
            pltpu.SemaphoreType.DMA,
            pltpu.SemaphoreType.DMA,
        ],
    )
    def k(tlon_hbm, tlat_hbm, ilon_hbm, ilat_hbm, out_hbm,
          tlon_v, tlat_v, ilon0, ilon1, ilat0, ilat1, obuf,
          si0, si1, so0, so1):
        si = (si0, si1)
        so = (so0, so1)
        ilon = (ilon0, ilon1)
        ilat = (ilat0, ilat1)
        wid = lax.axis_index("s") * _NC + lax.axis_index("c")
        base = wid * _BPW

        pltpu.sync_copy(tlon_hbm, tlon_v)
        pltpu.sync_copy(tlat_hbm, tlat_v)

        def idx_src(c):
            off = pl.multiple_of(base + c * _CHUNK, _CHUNK)
            return (ilon_hbm.at[pl.ds(off, _CHUNK)],
                    ilat_hbm.at[pl.ds(off, _CHUNK)])

        def out_dst(c):
            return out_hbm.at[wid * _NCHUNK + c]

        for b in (0, 1):
            slon, slat = idx_src(b)
            pltpu.async_copy(slon, ilon[b], si[b])
            pltpu.async_copy(slat, ilat[b], si[b])

        @pl.loop(0, _NCHUNK, step=2)
        def _pair(c0):
            for b in (0, 1):
                c = c0 + b
                slon, slat = idx_src(c)
                pltpu.make_async_copy(slon, ilon[b], si[b]).wait()
                pltpu.make_async_copy(slat, ilat[b], si[b]).wait()

                @pl.when(c >= 2)
                def _drain():
                    pltpu.make_async_copy(obuf.at[b], out_dst(c - 2),
                                          so[b]).wait()

                @plsc.parallel_loop(0, _GRP, 1, unroll=2)
                def _grp(g):
                    ivl = ilon[b][pl.ds(g * 16, 16)]
                    ivt = ilat[b][pl.ds(g * 16, 16)]
                    for d in range(EMB_DIM):
                        gl = plsc.load_gather(tlon_v, [ivl + d * _V])
                        gt = plsc.load_gather(tlat_v, [ivt + d * _V])
                        obuf[b, d, pl.ds(g * 16, 16)] = gl + gt

                pltpu.async_copy(obuf.at[b], out_dst(c), so[b])

                @pl.when(c + 2 < _NCHUNK)
                def _prefetch():
                    nlon, nlat = idx_src(c + 2)
                    pltpu.async_copy(nlon, ilon[b], si[b])
                    pltpu.async_copy(nlat, ilat[b], si[b])

        for b in (0, 1):
            pltpu.make_async_copy(obuf.at[b], out_dst(_NCHUNK - 2 + b),
                                  so[b]).wait()

    return k(tlon_flat, tlat_flat, idx_lon, idx_lat)


def kernel(d_mat_lon, d_mat_lat, table_lon, table_lat):
    idx_lon, idx_lat = _compute_indices(d_mat_lon, d_mat_lat)
    tlon_flat = jnp.transpose(table_lon).reshape(-1)
    tlat_flat = jnp.transpose(table_lat).reshape(-1)
    out_acb = _sc_gather_sum(tlon_flat, tlat_flat,
                             idx_lon.reshape(-1), idx_lat.reshape(-1))
    return jnp.transpose(out_acb, (0, 2, 1))
